# baseline (speedup 1.0000x reference)
import jax
import jax.numpy as jnp
import numpy as np
from jax import lax
from jax.experimental import pallas as pl
from jax.experimental.pallas import tpu as pltpu
from jax.experimental.pallas import tpu_sc as plsc

VOCAB = 100000
EMBED = 128
BATCH = 4096
HIST = 200

NWORKERS = 32
BPW = BATCH // NWORKERS
NCHUNK = (HIST + 15) // 16
NTAIL = HIST - (NCHUNK - 1) * 16
VBLK = 2000


def _repack_body(w_ref, y_ref):
    xu = w_ref[...].astype(jnp.int32) & np.int32(0xFFFF)
    xn = jnp.concatenate([xu[:, 1:], xu[:, :1]], axis=1)
    y_ref[...] = xu | lax.shift_left(xn, 16)


def _sc_body(ids_hbm, y_hbm, out_hbm, ids_all, rows_v, out_v,
             gsem0, gsem1, osem):
    wid = lax.axis_index("s") * 2 + lax.axis_index("c")
    base_b = wid * BPW
    gsems = (gsem0, gsem1)

    iota = lax.iota(jnp.int32, 16)
    l_idx = [jnp.minimum(lc * 16 + iota, HIST - 1) for lc in range(NCHUNK)]
    tail_mask = iota < NTAIL

    pltpu.sync_copy(ids_hbm.at[pl.ds(base_b, BPW)], ids_all)

    def prepare(bn, buf):
        for r in range(2):
            pltpu.async_copy(
                y_hbm.at[ids_all.at[bn, r]],
                rows_v.at[buf, pl.ds(r * 100, 100)], gsems[buf])

    def transpose(buf):
        for c in range(4):
            colv = 32 * c + 2 * iota
            rowv = 16 * c + iota

            @plsc.parallel_loop(0, NCHUNK, unroll=1)
            def per_group(g):
                for o in range(16):
                    lraw = g * 16 + ((iota + o) & 15)
                    mask = lraw < HIST
                    lvec = jnp.minimum(lraw, HIST - 1)
                    word = plsc.load_gather(rows_v.at[buf],
                                            [lvec, colv])
                    plsc.store_scatter(out_v.at[buf], [rowv, lvec],
                                       word, mask=mask)

    def wait_gather(buf):
        pltpu.make_async_copy(
            y_hbm.at[pl.ds(0, HIST)], rows_v.at[buf], gsems[buf]).wait()

    def drain_out(buf):
        pltpu.make_async_copy(
            out_hbm.at[base_b], out_v.at[buf].bitcast(jnp.float16),
            osem).wait()

    prepare(0, 0)

    def step(k, _):
        for buf in range(2):
            bi = 2 * k + buf
            prepare(jnp.minimum(bi + 1, BPW - 1), 1 - buf)
            wait_gather(buf)

            @pl.when(k >= 1)
            def _drain():
                drain_out(buf)

            transpose(buf)
            pltpu.async_copy(
                out_v.at[buf].bitcast(jnp.float16),
                out_hbm.at[base_b + bi], osem)
        return _

    lax.fori_loop(0, BPW // 2, step, None)

    wait_gather(0)
    drain_out(0)
    drain_out(1)


@jax.jit
def kernel(ids, weight):
    ids3 = ids.astype(jnp.int32).reshape(BATCH, 2, HIST // 2)

    wu = lax.bitcast_convert_type(weight, jnp.uint16)
    y = pl.pallas_call(
        _repack_body,
        grid=(VOCAB // VBLK,),
        in_specs=[pl.BlockSpec((VBLK, EMBED), lambda i: (i, 0))],
        out_specs=pl.BlockSpec((VBLK, EMBED), lambda i: (i, 0)),
        out_shape=jax.ShapeDtypeStruct((VOCAB, EMBED), jnp.int32),
    )(wu)

    mesh = plsc.VectorSubcoreMesh(core_axis_name="c", subcore_axis_name="s")
    out_sc = pl.kernel(
        _sc_body,
        out_type=jax.ShapeDtypeStruct((BATCH, EMBED, HIST), jnp.float16),
        mesh=mesh,
        scratch_types=[
            pltpu.VMEM((BPW, 2, HIST // 2), jnp.int32),
            pltpu.VMEM((2, HIST, EMBED), jnp.int32),
            pltpu.VMEM((2, EMBED // 2, HIST), jnp.int32),
            pltpu.SemaphoreType.DMA,
            pltpu.SemaphoreType.DMA,
            pltpu.SemaphoreType.DMA,
        ],
        compiler_params=pltpu.CompilerParams(needs_layout_passes=False),
    )(ids3, y)

    return out_sc


# speedup vs baseline: 1.0694x; 1.0694x over previous
import jax
import jax.numpy as jnp
import numpy as np
from jax import lax
from jax.experimental import pallas as pl
from jax.experimental.pallas import tpu as pltpu
from jax.experimental.pallas import tpu_sc as plsc

VOCAB = 100000
EMBED = 128
BATCH = 4096
HIST = 200

NWORKERS = 32
BPW = BATCH // NWORKERS
NCHUNK = (HIST + 15) // 16
NTAIL = HIST - (NCHUNK - 1) * 16
VBLK = 2000


def _repack_body(w_ref, y_ref):
    xu = w_ref[...].astype(jnp.int32) & np.int32(0xFFFF)
    xn = jnp.concatenate([xu[:, 1:], xu[:, :1]], axis=1)
    y_ref[...] = xu | lax.shift_left(xn, 16)


def _sc_body(ids_hbm, y_hbm, out_hbm, ids_all, rows_v, out_v,
             gsem0, gsem1, osem):
    wid = lax.axis_index("s") * 2 + lax.axis_index("c")
    base_b = wid * BPW
    gsems = (gsem0, gsem1)

    iota = lax.iota(jnp.int32, 16)
    l_idx = [jnp.minimum(lc * 16 + iota, HIST - 1) for lc in range(NCHUNK)]
    tail_mask = iota < NTAIL

    pltpu.sync_copy(ids_hbm.at[pl.ds(base_b, BPW)], ids_all)

    def prepare(bn, buf):
        for r in range(2):
            pltpu.async_copy(
                y_hbm.at[ids_all.at[bn, r]],
                rows_v.at[buf, pl.ds(r * 100, 100)], gsems[buf])

    def transpose(buf):
        for c in range(4):
            @plsc.parallel_loop(0, NCHUNK, unroll=1)
            def per_group(g):
                lvec_raw = g * 16 + iota
                mask = lvec_raw < HIST
                lvec = jnp.minimum(lvec_raw, HIST - 1)
                for o in range(16):
                    perm = (iota + o) & 15
                    colv = 32 * c + 2 * perm
                    word = plsc.load_gather(rows_v.at[buf], [lvec, colv])
                    plsc.store_scatter(out_v.at[buf],
                                       [16 * c + perm, lvec],
                                       word, mask=mask)

    def wait_gather(buf):
        pltpu.make_async_copy(
            y_hbm.at[pl.ds(0, HIST)], rows_v.at[buf], gsems[buf]).wait()

    def drain_out(buf):
        pltpu.make_async_copy(
            out_hbm.at[base_b], out_v.at[buf].bitcast(jnp.float16),
            osem).wait()

    prepare(0, 0)

    def step(k, _):
        for buf in range(2):
            bi = 2 * k + buf
            prepare(jnp.minimum(bi + 1, BPW - 1), 1 - buf)
            wait_gather(buf)

            @pl.when(k >= 1)
            def _drain():
                drain_out(buf)

            transpose(buf)
            pltpu.async_copy(
                out_v.at[buf].bitcast(jnp.float16),
                out_hbm.at[base_b + bi], osem)
        return _

    lax.fori_loop(0, BPW // 2, step, None)

    wait_gather(0)
    drain_out(0)
    drain_out(1)


@jax.jit
def kernel(ids, weight):
    ids3 = ids.astype(jnp.int32).reshape(BATCH, 2, HIST // 2)

    wu = lax.bitcast_convert_type(weight, jnp.uint16)
    y = pl.pallas_call(
        _repack_body,
        grid=(VOCAB // VBLK,),
        in_specs=[pl.BlockSpec((VBLK, EMBED), lambda i: (i, 0))],
        out_specs=pl.BlockSpec((VBLK, EMBED), lambda i: (i, 0)),
        out_shape=jax.ShapeDtypeStruct((VOCAB, EMBED), jnp.int32),
    )(wu)

    mesh = plsc.VectorSubcoreMesh(core_axis_name="c", subcore_axis_name="s")
    out_sc = pl.kernel(
        _sc_body,
        out_type=jax.ShapeDtypeStruct((BATCH, EMBED, HIST), jnp.float16),
        mesh=mesh,
        scratch_types=[
            pltpu.VMEM((BPW, 2, HIST // 2), jnp.int32),
            pltpu.VMEM((2, HIST, EMBED), jnp.int32),
            pltpu.VMEM((2, EMBED // 2, HIST), jnp.int32),
            pltpu.SemaphoreType.DMA,
            pltpu.SemaphoreType.DMA,
            pltpu.SemaphoreType.DMA,
        ],
        compiler_params=pltpu.CompilerParams(needs_layout_passes=False),
    )(ids3, y)

    return out_sc
